# baseline (speedup 1.0000x reference)
import functools
import math

import jax
import jax.numpy as jnp
from jax import lax
from jax.experimental import pallas as pl
from jax.experimental.pallas import tpu as pltpu
from jax.experimental.pallas import tpu_sc as plsc

EMB = 64
SCALE = math.sqrt(EMB)

NC = 2
NS = 16
NW = NC * NS

G = 128
NBUF = 4
BV = 4096
SPLIT = 123


def _transpose_body(x1_ref, x2_ref, o_ref):
    ii = lax.broadcasted_iota(jnp.int32, (EMB, EMB), 0)
    jj = lax.broadcasted_iota(jnp.int32, (EMB, EMB), 1)
    seye = jnp.where(ii == jj, SCALE, 0.0).astype(jnp.float32)
    dims = (((0,), (0,)), ((), ()))
    o_ref[:, 0:EMB] = lax.dot_general(
        x1_ref[...], seye, dims, precision=lax.Precision.HIGHEST,
        preferred_element_type=jnp.float32)
    o_ref[:, EMB:2 * EMB] = lax.dot_general(
        x2_ref[...], seye, dims, precision=lax.Precision.HIGHEST,
        preferred_element_type=jnp.float32)


def _gather_body(tok_hbm, table_hbm, out_hbm, idx_v, *rows_and_sems):
    rows = rows_and_sems[:NBUF]
    sems = rows_and_sems[NBUF:]
    ng = tok_hbm.shape[1]
    per_w = ng * G

    wid = lax.axis_index("s") * NC + lax.axis_index("c")
    base = wid * per_w

    pltpu.sync_copy(tok_hbm.at[wid], idx_v)

    for b in range(NBUF):
        pltpu.async_copy(table_hbm.at[idx_v.at[b]], rows[b], sems[b])

    @pl.loop(0, ng, step=NBUF)
    def _chunks(t):
        for b in range(NBUF):
            g = t + b
            pltpu.make_async_copy(
                table_hbm.at[idx_v.at[g]], rows[b], sems[b]
            ).wait()

            pltpu.sync_copy(
                rows[b],
                out_hbm.at[pl.ds(base + g * G, G), pl.ds(0, EMB)],
            )

            gn = g + NBUF

            @pl.when(gn < ng)
            def _():
                pltpu.async_copy(table_hbm.at[idx_v.at[gn]], rows[b], sems[b])


def kernel(tokens, table):
    n_tok = tokens.shape[0] * tokens.shape[1]
    per_w = n_tok // NW
    ng = per_w // G
    vocab = table.shape[0]
    nblk = (vocab + BV - 1) // BV
    cut = SPLIT * BV

    transpose_run = pl.pallas_call(
        _transpose_body,
        grid=(SPLIT,),
        in_specs=[
            pl.BlockSpec((EMB, BV), lambda i: (0, i)),
            pl.BlockSpec(
                (EMB, BV),
                lambda i: (0, jnp.minimum(i + SPLIT, nblk - 1)),
            ),
        ],
        out_specs=pl.BlockSpec((BV, 2 * EMB), lambda i: (i, 0)),
        out_shape=jax.ShapeDtypeStruct((cut, 2 * EMB), jnp.float32),
    )

    mesh = plsc.VectorSubcoreMesh(core_axis_name="c", subcore_axis_name="s")
    gather_run = pl.kernel(
        _gather_body,
        out_type=jax.ShapeDtypeStruct((n_tok, 2 * EMB), jnp.float32),
        mesh=mesh,
        compiler_params=pltpu.CompilerParams(use_tc_tiling_on_sc=False),
        scratch_types=(
            [pltpu.VMEM((ng, G), jnp.int32)]
            + [pltpu.VMEM((G, EMB), jnp.float32) for _ in range(NBUF)]
            + [pltpu.SemaphoreType.DMA for _ in range(NBUF)]
        ),
    )

    tt = table.T
    packed = transpose_run(tt, tt)
    tclean = packed.reshape(2 * cut, EMB)

    tok = tokens.astype(jnp.int32)
    idx = jnp.where(tok < cut, 2 * tok, 2 * (tok - cut) + 1)
    idx = idx.reshape(NW, ng, G)

    out = gather_run(idx, tclean)
    return out[:, :EMB].reshape(tokens.shape[0], tokens.shape[1], EMB)


# speedup vs baseline: 1.4095x; 1.4095x over previous
import functools
import math

import jax
import jax.numpy as jnp
from jax import lax
from jax.experimental import pallas as pl
from jax.experimental.pallas import tpu as pltpu
from jax.experimental.pallas import tpu_sc as plsc

EMB = 64
SCALE = math.sqrt(EMB)

NC = 2
NS = 16
NW = NC * NS

G = 128
NBUF = 4
BV = 4096
SPLIT = 123


def _transpose_body(x1_ref, x2_ref, o_ref):
    ii = lax.broadcasted_iota(jnp.int32, (EMB, EMB), 0)
    jj = lax.broadcasted_iota(jnp.int32, (EMB, EMB), 1)
    seye = jnp.where(ii == jj, SCALE, 0.0).astype(jnp.float32)
    dims = (((0,), (0,)), ((), ()))
    o_ref[:, 0:EMB] = lax.dot_general(
        x1_ref[...], seye, dims, preferred_element_type=jnp.float32)
    o_ref[:, EMB:2 * EMB] = lax.dot_general(
        x2_ref[...], seye, dims, preferred_element_type=jnp.float32)


def _gather_body(tok_hbm, table_hbm, out_hbm, idx_v, *rows_and_sems):
    rows = rows_and_sems[:NBUF]
    sems = rows_and_sems[NBUF:]
    ng = tok_hbm.shape[1]
    per_w = ng * G

    wid = lax.axis_index("s") * NC + lax.axis_index("c")
    base = wid * per_w

    pltpu.sync_copy(tok_hbm.at[wid], idx_v)

    for b in range(NBUF):
        pltpu.async_copy(table_hbm.at[idx_v.at[b]], rows[b], sems[b])

    @pl.loop(0, ng, step=NBUF)
    def _chunks(t):
        for b in range(NBUF):
            g = t + b
            pltpu.make_async_copy(
                table_hbm.at[idx_v.at[g]], rows[b], sems[b]
            ).wait()

            pltpu.sync_copy(
                rows[b],
                out_hbm.at[pl.ds(base + g * G, G), pl.ds(0, EMB)],
            )

            gn = g + NBUF

            @pl.when(gn < ng)
            def _():
                pltpu.async_copy(table_hbm.at[idx_v.at[gn]], rows[b], sems[b])


def kernel(tokens, table):
    n_tok = tokens.shape[0] * tokens.shape[1]
    per_w = n_tok // NW
    ng = per_w // G
    vocab = table.shape[0]
    nblk = (vocab + BV - 1) // BV
    cut = SPLIT * BV

    transpose_run = pl.pallas_call(
        _transpose_body,
        grid=(SPLIT,),
        in_specs=[
            pl.BlockSpec((EMB, BV), lambda i: (0, i)),
            pl.BlockSpec(
                (EMB, BV),
                lambda i: (0, jnp.minimum(i + SPLIT, nblk - 1)),
            ),
        ],
        out_specs=pl.BlockSpec((BV, 2 * EMB), lambda i: (i, 0)),
        out_shape=jax.ShapeDtypeStruct((cut, 2 * EMB), jnp.float32),
    )

    mesh = plsc.VectorSubcoreMesh(core_axis_name="c", subcore_axis_name="s")
    gather_run = pl.kernel(
        _gather_body,
        out_type=jax.ShapeDtypeStruct((n_tok, 2 * EMB), jnp.float32),
        mesh=mesh,
        compiler_params=pltpu.CompilerParams(use_tc_tiling_on_sc=False),
        scratch_types=(
            [pltpu.VMEM((ng, G), jnp.int32)]
            + [pltpu.VMEM((G, EMB), jnp.float32) for _ in range(NBUF)]
            + [pltpu.SemaphoreType.DMA for _ in range(NBUF)]
        ),
    )

    tt = table.T
    packed = transpose_run(tt, tt)
    tclean = packed.reshape(2 * cut, EMB)

    tok = tokens.astype(jnp.int32)
    idx = jnp.where(tok < cut, 2 * tok, 2 * (tok - cut) + 1)
    idx = idx.reshape(NW, ng, G)

    out = gather_run(idx, tclean)
    return out[:, :EMB].reshape(tokens.shape[0], tokens.shape[1], EMB)


# speedup vs baseline: 1.4903x; 1.0574x over previous
import functools
import math

import jax
import jax.numpy as jnp
from jax import lax
from jax.experimental import pallas as pl
from jax.experimental.pallas import tpu as pltpu
from jax.experimental.pallas import tpu_sc as plsc

EMB = 64
SCALE = math.sqrt(EMB)

NC = 2
NS = 16
NW = NC * NS

G = 128
NBUF = 4
BV = 8192
SPLIT = 62


def _transpose_body(x1_ref, x2_ref, o_ref):
    o_ref[:, 0:EMB] = (x1_ref[...] * SCALE).T
    o_ref[:, EMB:2 * EMB] = (x2_ref[...] * SCALE).T


def _gather_body(tok_hbm, table_hbm, out_hbm, idx_v, *rows_and_sems):
    rows = rows_and_sems[:NBUF]
    sems = rows_and_sems[NBUF:]
    ng = tok_hbm.shape[1]
    per_w = ng * G

    wid = lax.axis_index("s") * NC + lax.axis_index("c")
    base = wid * per_w

    pltpu.sync_copy(tok_hbm.at[wid], idx_v)

    for b in range(NBUF):
        pltpu.async_copy(table_hbm.at[idx_v.at[b]], rows[b], sems[b])

    @pl.loop(0, ng, step=NBUF)
    def _chunks(t):
        for b in range(NBUF):
            g = t + b
            pltpu.make_async_copy(
                table_hbm.at[idx_v.at[g]], rows[b], sems[b]
            ).wait()

            pltpu.sync_copy(
                rows[b],
                out_hbm.at[pl.ds(base + g * G, G), pl.ds(0, EMB)],
            )

            gn = g + NBUF

            @pl.when(gn < ng)
            def _():
                pltpu.async_copy(table_hbm.at[idx_v.at[gn]], rows[b], sems[b])


def kernel(tokens, table):
    n_tok = tokens.shape[0] * tokens.shape[1]
    per_w = n_tok // NW
    ng = per_w // G
    vocab = table.shape[0]
    nblk = (vocab + BV - 1) // BV
    cut = SPLIT * BV

    transpose_run = pl.pallas_call(
        _transpose_body,
        grid=(SPLIT,),
        in_specs=[
            pl.BlockSpec((EMB, BV), lambda i: (0, i)),
            pl.BlockSpec(
                (EMB, BV),
                lambda i: (0, jnp.minimum(i + SPLIT, nblk - 1)),
            ),
        ],
        out_specs=pl.BlockSpec((BV, 2 * EMB), lambda i: (i, 0)),
        out_shape=jax.ShapeDtypeStruct((cut, 2 * EMB), jnp.float32),
    )

    mesh = plsc.VectorSubcoreMesh(core_axis_name="c", subcore_axis_name="s")
    gather_run = pl.kernel(
        _gather_body,
        out_type=jax.ShapeDtypeStruct((n_tok, 2 * EMB), jnp.float32),
        mesh=mesh,
        compiler_params=pltpu.CompilerParams(use_tc_tiling_on_sc=False),
        scratch_types=(
            [pltpu.VMEM((ng, G), jnp.int32)]
            + [pltpu.VMEM((G, EMB), jnp.float32) for _ in range(NBUF)]
            + [pltpu.SemaphoreType.DMA for _ in range(NBUF)]
        ),
    )

    tt = table.T
    packed = transpose_run(tt, tt)
    tclean = packed.reshape(2 * cut, EMB)

    tok = tokens.astype(jnp.int32)
    idx = jnp.where(tok < cut, 2 * tok, 2 * (tok - cut) + 1)
    idx = idx.reshape(NW, ng, G)

    out = gather_run(idx, tclean)
    return out[:, :EMB].reshape(tokens.shape[0], tokens.shape[1], EMB)


# speedup vs baseline: 1.5243x; 1.0228x over previous
import functools
import math

import jax
import jax.numpy as jnp
from jax import lax
from jax.experimental import pallas as pl
from jax.experimental.pallas import tpu as pltpu
from jax.experimental.pallas import tpu_sc as plsc

EMB = 64
SCALE = math.sqrt(EMB)

NC = 2
NS = 16
NW = NC * NS

G = 128
NBUF = 4
BV = 16384
SPLIT = 31


def _transpose_body(x1_ref, x2_ref, o_ref):
    o_ref[:, 0:EMB] = (x1_ref[...] * SCALE).T
    o_ref[:, EMB:2 * EMB] = (x2_ref[...] * SCALE).T


def _gather_body(tok_hbm, table_hbm, out_hbm, idx_v, *rows_and_sems):
    rows = rows_and_sems[:NBUF]
    sems = rows_and_sems[NBUF:]
    ng = tok_hbm.shape[1]
    per_w = ng * G

    wid = lax.axis_index("s") * NC + lax.axis_index("c")
    base = wid * per_w

    pltpu.sync_copy(tok_hbm.at[wid], idx_v)

    for b in range(NBUF):
        pltpu.async_copy(table_hbm.at[idx_v.at[b]], rows[b], sems[b])

    @pl.loop(0, ng, step=NBUF)
    def _chunks(t):
        for b in range(NBUF):
            g = t + b
            pltpu.make_async_copy(
                table_hbm.at[idx_v.at[g]], rows[b], sems[b]
            ).wait()

            pltpu.sync_copy(
                rows[b],
                out_hbm.at[pl.ds(base + g * G, G), pl.ds(0, EMB)],
            )

            gn = g + NBUF

            @pl.when(gn < ng)
            def _():
                pltpu.async_copy(table_hbm.at[idx_v.at[gn]], rows[b], sems[b])


def kernel(tokens, table):
    n_tok = tokens.shape[0] * tokens.shape[1]
    per_w = n_tok // NW
    ng = per_w // G
    vocab = table.shape[0]
    nblk = (vocab + BV - 1) // BV
    cut = SPLIT * BV

    transpose_run = pl.pallas_call(
        _transpose_body,
        grid=(SPLIT,),
        in_specs=[
            pl.BlockSpec((EMB, BV), lambda i: (0, i)),
            pl.BlockSpec(
                (EMB, BV),
                lambda i: (0, jnp.minimum(i + SPLIT, nblk - 1)),
            ),
        ],
        out_specs=pl.BlockSpec((BV, 2 * EMB), lambda i: (i, 0)),
        out_shape=jax.ShapeDtypeStruct((cut, 2 * EMB), jnp.float32),
    )

    mesh = plsc.VectorSubcoreMesh(core_axis_name="c", subcore_axis_name="s")
    gather_run = pl.kernel(
        _gather_body,
        out_type=jax.ShapeDtypeStruct((n_tok, 2 * EMB), jnp.float32),
        mesh=mesh,
        compiler_params=pltpu.CompilerParams(use_tc_tiling_on_sc=False),
        scratch_types=(
            [pltpu.VMEM((ng, G), jnp.int32)]
            + [pltpu.VMEM((G, EMB), jnp.float32) for _ in range(NBUF)]
            + [pltpu.SemaphoreType.DMA for _ in range(NBUF)]
        ),
    )

    tt = table.T
    packed = transpose_run(tt, tt)
    tclean = packed.reshape(2 * cut, EMB)

    tok = tokens.astype(jnp.int32)
    idx = jnp.where(tok < cut, 2 * tok, 2 * (tok - cut) + 1)
    idx = idx.reshape(NW, ng, G)

    out = gather_run(idx, tclean)
    return out[:, :EMB].reshape(tokens.shape[0], tokens.shape[1], EMB)
